# baseline (speedup 1.0000x reference)
import functools

import jax
import jax.numpy as jnp
from jax import lax
from jax.experimental import pallas as pl
from jax.experimental.pallas import tpu as pltpu
from jax.experimental.pallas import tpu_sc as plsc

_NC = 2
_NS = 16
_L = 16


def _make_degree_kernel(N, E, NP):
    EPC = E // _NC
    EPW = EPC // _NS
    KH = 80
    n_chunks = EPW // KH
    KB = 25
    RPT = NP // _NS
    ZR = 128
    DL = _L
    mesh = plsc.VectorSubcoreMesh(core_axis_name="c", subcore_axis_name="s")

    @functools.partial(
        pl.kernel,
        out_type=jax.ShapeDtypeStruct((2 * _NC, NP, DL), jnp.float32),
        mesh=mesh,
        scratch_types=[
            [pltpu.VMEM((KH,), jnp.int32)] * KB,
            [pltpu.VMEM((KH,), jnp.int32)] * KB,
            pltpu.VMEM((KH, DL), jnp.float32),
            pltpu.VMEM((ZR, DL), jnp.float32),
            pltpu.VMEM_SHARED((NP, DL), jnp.float32),
            pltpu.VMEM_SHARED((NP, DL), jnp.float32),
            [pltpu.SemaphoreType.DMA] * KB,
        ],
    )
    def deg_kernel(src_hbm, dst_hbm, const_hbm, out_hbm,
                   sbufs, dbufs, ones_v, zbuf_v, acc_s, acc_d, sems):
        c = lax.axis_index("c")
        s = lax.axis_index("s")
        wid = c * _NS + s
        pltpu.sync_copy(const_hbm.at[pl.ds(0, ZR)], zbuf_v)
        pltpu.sync_copy(const_hbm.at[pl.ds(ZR, KH)], ones_v)

        r0 = s * RPT

        def zcopy(i, carry):
            pltpu.sync_copy(zbuf_v, acc_s.at[pl.ds(r0 + i * ZR, ZR)])
            pltpu.sync_copy(zbuf_v, acc_d.at[pl.ds(r0 + i * ZR, ZR)])
            return carry

        lax.fori_loop(0, RPT // ZR, zcopy, 0)
        plsc.subcore_barrier()

        base = wid * EPW

        def block(g, carry):
            for b in range(KB):
                off = base + (g * KB + b) * KH
                pltpu.async_copy(src_hbm.at[pl.ds(off, KH)], sbufs[b], sems[b])
                pltpu.async_copy(dst_hbm.at[pl.ds(off, KH)], dbufs[b], sems[b])
            for b in range(KB):
                off = base + (g * KB + b) * KH
                pltpu.make_async_copy(src_hbm.at[pl.ds(off, KH)], sbufs[b], sems[b]).wait()
                pltpu.make_async_copy(dst_hbm.at[pl.ds(off, KH)], dbufs[b], sems[b]).wait()
                pltpu.async_copy(ones_v, acc_s.at[sbufs[b]], sems[b], add=True)
                pltpu.async_copy(ones_v, acc_d.at[dbufs[b]], sems[b], add=True)
            for b in range(KB):
                pltpu.make_async_copy(ones_v, acc_s.at[sbufs[b]], sems[b]).wait()
                pltpu.make_async_copy(ones_v, acc_d.at[dbufs[b]], sems[b]).wait()
            return carry

        lax.fori_loop(0, n_chunks // KB, block, 0)
        plsc.subcore_barrier()

        def wcopy(i, carry):
            pltpu.sync_copy(acc_s.at[pl.ds(r0 + i * ZR, ZR)], zbuf_v)
            pltpu.sync_copy(zbuf_v, out_hbm.at[2 * c].at[pl.ds(r0 + i * ZR, ZR)])
            pltpu.sync_copy(acc_d.at[pl.ds(r0 + i * ZR, ZR)], zbuf_v)
            pltpu.sync_copy(zbuf_v, out_hbm.at[2 * c + 1].at[pl.ds(r0 + i * ZR, ZR)])
            return carry

        lax.fori_loop(0, RPT // ZR, wcopy, 0)

    return deg_kernel


def _make_segsum_kernel(E, D, NP):
    EPW = E // _NS
    CH = 80
    n_chunks = EPW // CH
    KB = 10 if D >= 64 else 25
    RPT = NP // _NS
    ZR = 128
    mesh = plsc.VectorSubcoreMesh(core_axis_name="c", subcore_axis_name="s")

    @functools.partial(
        pl.kernel,
        out_type=jax.ShapeDtypeStruct((_NC, NP, D), jnp.float32),
        mesh=mesh,
        scratch_types=[
            [pltpu.VMEM((CH,), jnp.int32)] * KB,
            [pltpu.VMEM((CH,), jnp.int32)] * KB,
            [pltpu.VMEM((CH, D), jnp.float32)] * KB,
            pltpu.VMEM((ZR, D), jnp.float32),
            pltpu.VMEM_SHARED((NP, D), jnp.float32),
            [pltpu.SemaphoreType.DMA] * KB,
        ],
        compiler_params=pltpu.CompilerParams(use_tc_tiling_on_sc=False),
    )
    def segsum_kernel(rows_hbm, src2_hbm, dst_hbm, zeros_hbm, out_hbm,
                      sbufs, dbufs, rbufs, zbuf_v, acc, sems):
        c = lax.axis_index("c")
        s = lax.axis_index("s")
        pltpu.sync_copy(zeros_hbm, zbuf_v)

        r0 = s * RPT

        def zcopy(i, carry):
            pltpu.sync_copy(zbuf_v, acc.at[pl.ds(r0 + i * ZR, ZR)])
            return carry

        lax.fori_loop(0, RPT // ZR, zcopy, 0)
        plsc.subcore_barrier()

        base = s * EPW

        def block(g, carry):
            for b in range(KB):
                off = base + (g * KB + b) * CH
                pltpu.async_copy(src2_hbm.at[c].at[pl.ds(off, CH)], sbufs[b], sems[b])
                pltpu.async_copy(dst_hbm.at[pl.ds(off, CH)], dbufs[b], sems[b])
            for b in range(KB):
                off = base + (g * KB + b) * CH
                pltpu.make_async_copy(src2_hbm.at[c].at[pl.ds(off, CH)], sbufs[b], sems[b]).wait()
                pltpu.make_async_copy(dst_hbm.at[pl.ds(off, CH)], dbufs[b], sems[b]).wait()
                pltpu.async_copy(rows_hbm.at[sbufs[b]], rbufs[b], sems[b])
            for b in range(KB):
                pltpu.make_async_copy(rows_hbm.at[sbufs[b]], rbufs[b], sems[b]).wait()
                pltpu.async_copy(rbufs[b], acc.at[dbufs[b]], sems[b], add=True)
            for b in range(KB):
                pltpu.make_async_copy(rbufs[b], acc.at[dbufs[b]], sems[b]).wait()
            return carry

        lax.fori_loop(0, n_chunks // KB, block, 0)
        plsc.subcore_barrier()

        def wcopy(i, carry):
            pltpu.sync_copy(acc.at[pl.ds(r0 + i * ZR, ZR)], zbuf_v)
            pltpu.sync_copy(zbuf_v, out_hbm.at[c].at[pl.ds(r0 + i * ZR, ZR)])
            return carry

        lax.fori_loop(0, RPT // ZR, wcopy, 0)

    return segsum_kernel


def _tc_norm_matmul(x, W1, deg_parts, edge_index, R=1000):
    N, D_IN = x.shape
    D_H = W1.shape[1]
    DL = deg_parts.shape[2]
    E = edge_index.shape[1]
    EB = E // (N // R)

    def body(x_ref, w_ref, deg_ref, ei_ref, xw_ref, nrm_ref, s2_ref):
        d = deg_ref[...]
        out_deg = jnp.sum(d[0] + d[2], axis=1, keepdims=True) * (1.0 / DL)
        in_deg = jnp.sum(d[1] + d[3], axis=1, keepdims=True) * (1.0 / DL)
        deg = jnp.concatenate([out_deg, in_deg], axis=1)
        nrm = lax.rsqrt(jnp.maximum(deg, 1.0))
        nrm_ref[...] = nrm
        out_n = nrm[:, 0:1]
        xw = (
            jnp.dot(x_ref[...], w_ref[...], preferred_element_type=jnp.float32)
            * out_n
        )
        h = xw.shape[1] // 2
        xw_ref[0] = xw[:, :h]
        xw_ref[1] = xw[:, h:]
        s = ei_ref[0]
        s2_ref[0, :] = s
        s2_ref[1, :] = s + x_ref.shape[0] * pl.num_programs(0)

    return pl.pallas_call(
        body,
        grid=(N // R,),
        in_specs=[
            pl.BlockSpec((R, D_IN), lambda i: (i, 0)),
            pl.BlockSpec((D_IN, D_H), lambda i: (0, 0)),
            pl.BlockSpec((4, R, DL), lambda i: (0, i, 0)),
            pl.BlockSpec((2, EB), lambda i: (0, i)),
        ],
        out_specs=[
            pl.BlockSpec((2, R, D_H // 2), lambda i: (0, i, 0)),
            pl.BlockSpec((R, 2), lambda i: (i, 0)),
            pl.BlockSpec((2, EB), lambda i: (0, i)),
        ],
        out_shape=[
            jax.ShapeDtypeStruct((2, N, D_H // 2), jnp.float32),
            jax.ShapeDtypeStruct((N, 2), jnp.float32),
            jax.ShapeDtypeStruct((2, E), jnp.int32),
        ],
    )(x, W1, deg_parts, edge_index)


def _tc_mid(parts, norms, b1, W2p, R=1000):
    N = norms.shape[0]
    DHH = parts.shape[2]
    D_H = 2 * DHH
    D2 = W2p.shape[1]

    def body(p_ref, n_ref, b_ref, w_ref, o_ref):
        m = jnp.concatenate([p_ref[0], p_ref[1]], axis=1)
        nrm = n_ref[...]
        h = jnp.maximum(m * nrm[:, 1:2] + b_ref[...], 0.0)
        h2 = (
            jnp.dot(h, w_ref[...], preferred_element_type=jnp.float32)
            * nrm[:, 0:1]
        )
        hh = h2.shape[1] // 2
        o_ref[0] = h2[:, :hh]
        o_ref[1] = h2[:, hh:]

    return pl.pallas_call(
        body,
        grid=(N // R,),
        in_specs=[
            pl.BlockSpec((2, R, DHH), lambda i: (0, i, 0)),
            pl.BlockSpec((R, 2), lambda i: (i, 0)),
            pl.BlockSpec((1, D_H), lambda i: (0, 0)),
            pl.BlockSpec((D_H, D2), lambda i: (0, 0)),
        ],
        out_specs=pl.BlockSpec((2, R, D2 // 2), lambda i: (0, i, 0)),
        out_shape=jax.ShapeDtypeStruct((2, N, D2 // 2), jnp.float32),
    )(parts, norms, b1, W2p)


def _tc_final(parts, norms, b2p, NC_OUT, R=1000):
    N = norms.shape[0]
    D2 = 2 * parts.shape[2]

    def body(p_ref, n_ref, b_ref, o_ref):
        m = jnp.concatenate([p_ref[0], p_ref[1]], axis=1)
        o = m * n_ref[:, 1:2] + b_ref[...]
        o_ref[...] = o[:, :NC_OUT]

    return pl.pallas_call(
        body,
        grid=(N // R,),
        in_specs=[
            pl.BlockSpec((2, R, D2 // 2), lambda i: (0, i, 0)),
            pl.BlockSpec((R, 2), lambda i: (i, 0)),
            pl.BlockSpec((1, D2), lambda i: (0, 0)),
        ],
        out_specs=pl.BlockSpec((R, NC_OUT), lambda i: (i, 0)),
        out_shape=jax.ShapeDtypeStruct((N, NC_OUT), jnp.float32),
    )(parts, norms, b2p)


def kernel(x, edge_index, W1, b1, W2, b2):
    N, D_IN = x.shape
    E = edge_index.shape[1]
    D_H = W1.shape[1]
    N_CLS = W2.shape[1]
    D2 = 64

    src = edge_index[0]
    dst = edge_index[1]

    W2p = jnp.zeros((D_H, D2), jnp.float32).at[:, :N_CLS].set(W2)
    b2p = jnp.zeros((1, D2), jnp.float32).at[0, :N_CLS].set(b2)
    b1r = b1.reshape(1, D_H)

    NP = ((N + _NS * 128 - 1) // (_NS * 128)) * (_NS * 128)
    deg_const = jnp.concatenate(
        [jnp.zeros((128, _L), jnp.float32), jnp.ones((80, _L), jnp.float32)])
    deg_parts = _make_degree_kernel(N, E, NP)(src, dst, deg_const)

    xw3, norms, src2 = _tc_norm_matmul(x, W1, deg_parts, edge_index)
    xw2 = xw3.reshape(2 * N, D_H // 2)

    cols1 = _make_segsum_kernel(E, D_H // 2, NP)(
        xw2, src2, dst, jnp.zeros((128, D_H // 2), jnp.float32))
    h2w3 = _tc_mid(cols1, norms, b1r, W2p)
    h2w2 = h2w3.reshape(2 * N, D2 // 2)

    cols2 = _make_segsum_kernel(E, D2 // 2, NP)(
        h2w2, src2, dst, jnp.zeros((128, D2 // 2), jnp.float32))
    return _tc_final(cols2, norms, b2p, N_CLS)


# speedup vs baseline: 1.0511x; 1.0511x over previous
import functools

import jax
import jax.numpy as jnp
from jax import lax
from jax.experimental import pallas as pl
from jax.experimental.pallas import tpu as pltpu
from jax.experimental.pallas import tpu_sc as plsc

_NC = 2
_NS = 16
_L = 16


def _make_degree_kernel(N, E, NP):
    EPC = E // _NC
    EPW = EPC // _NS
    KH = 80
    n_chunks = EPW // KH
    KB = 25
    RPT = NP // _NS
    ZR = 128
    DL = _L
    mesh = plsc.VectorSubcoreMesh(core_axis_name="c", subcore_axis_name="s")

    @functools.partial(
        pl.kernel,
        out_type=jax.ShapeDtypeStruct((2 * _NC, NP, DL), jnp.float32),
        mesh=mesh,
        scratch_types=[
            [pltpu.VMEM((KH,), jnp.int32)] * KB,
            [pltpu.VMEM((KH,), jnp.int32)] * KB,
            pltpu.VMEM((KH, DL), jnp.float32),
            pltpu.VMEM((ZR, DL), jnp.float32),
            pltpu.VMEM_SHARED((NP, DL), jnp.float32),
            pltpu.VMEM_SHARED((NP, DL), jnp.float32),
            [pltpu.SemaphoreType.DMA] * KB,
        ],
    )
    def deg_kernel(src_hbm, dst_hbm, const_hbm, out_hbm,
                   sbufs, dbufs, ones_v, zbuf_v, acc_s, acc_d, sems):
        c = lax.axis_index("c")
        s = lax.axis_index("s")
        wid = c * _NS + s
        pltpu.sync_copy(const_hbm.at[pl.ds(0, ZR)], zbuf_v)
        pltpu.sync_copy(const_hbm.at[pl.ds(ZR, KH)], ones_v)

        r0 = s * RPT

        def zcopy(i, carry):
            pltpu.sync_copy(zbuf_v, acc_s.at[pl.ds(r0 + i * ZR, ZR)])
            pltpu.sync_copy(zbuf_v, acc_d.at[pl.ds(r0 + i * ZR, ZR)])
            return carry

        lax.fori_loop(0, RPT // ZR, zcopy, 0)
        plsc.subcore_barrier()

        base = wid * EPW

        def block(g, carry):
            for b in range(KB):
                off = base + (g * KB + b) * KH
                pltpu.async_copy(src_hbm.at[pl.ds(off, KH)], sbufs[b], sems[b])
                pltpu.async_copy(dst_hbm.at[pl.ds(off, KH)], dbufs[b], sems[b])
            for b in range(KB):
                off = base + (g * KB + b) * KH
                pltpu.make_async_copy(src_hbm.at[pl.ds(off, KH)], sbufs[b], sems[b]).wait()
                pltpu.make_async_copy(dst_hbm.at[pl.ds(off, KH)], dbufs[b], sems[b]).wait()
                pltpu.async_copy(ones_v, acc_s.at[sbufs[b]], sems[b], add=True)
                pltpu.async_copy(ones_v, acc_d.at[dbufs[b]], sems[b], add=True)
            for b in range(KB):
                pltpu.make_async_copy(ones_v, acc_s.at[sbufs[b]], sems[b]).wait()
                pltpu.make_async_copy(ones_v, acc_d.at[dbufs[b]], sems[b]).wait()
            return carry

        lax.fori_loop(0, n_chunks // KB, block, 0)
        plsc.subcore_barrier()

        def wcopy(i, carry):
            pltpu.sync_copy(acc_s.at[pl.ds(r0 + i * ZR, ZR)], zbuf_v)
            pltpu.sync_copy(zbuf_v, out_hbm.at[2 * c].at[pl.ds(r0 + i * ZR, ZR)])
            pltpu.sync_copy(acc_d.at[pl.ds(r0 + i * ZR, ZR)], zbuf_v)
            pltpu.sync_copy(zbuf_v, out_hbm.at[2 * c + 1].at[pl.ds(r0 + i * ZR, ZR)])
            return carry

        lax.fori_loop(0, RPT // ZR, wcopy, 0)

    return deg_kernel


def _make_segsum_kernel(E, D, NP):
    EPW = E // _NS
    CH = 80
    n_chunks = EPW // CH
    KB = 10 if D >= 64 else 25
    RPT = NP // _NS
    ZR = 128
    mesh = plsc.VectorSubcoreMesh(core_axis_name="c", subcore_axis_name="s")

    @functools.partial(
        pl.kernel,
        out_type=jax.ShapeDtypeStruct((_NC, NP, D), jnp.float32),
        mesh=mesh,
        scratch_types=[
            [pltpu.VMEM((CH,), jnp.int32)] * KB,
            [pltpu.VMEM((CH,), jnp.int32)] * KB,
            [pltpu.VMEM((CH, D), jnp.float32)] * KB,
            pltpu.VMEM((ZR, D), jnp.float32),
            pltpu.VMEM_SHARED((NP, D), jnp.float32),
            [pltpu.SemaphoreType.DMA] * KB,
        ],
        compiler_params=pltpu.CompilerParams(use_tc_tiling_on_sc=False),
    )
    def segsum_kernel(rows_hbm, src2_hbm, dst_hbm, zeros_hbm, out_hbm,
                      sbufs, dbufs, rbufs, zbuf_v, acc, sems):
        c = lax.axis_index("c")
        s = lax.axis_index("s")
        pltpu.sync_copy(zeros_hbm, zbuf_v)

        r0 = s * RPT

        def zcopy(i, carry):
            pltpu.sync_copy(zbuf_v, acc.at[pl.ds(r0 + i * ZR, ZR)])
            return carry

        lax.fori_loop(0, RPT // ZR, zcopy, 0)
        plsc.subcore_barrier()

        base = s * EPW

        def block(g, carry):
            for b in range(KB):
                off = base + (g * KB + b) * CH
                pltpu.async_copy(src2_hbm.at[c].at[pl.ds(off, CH)], sbufs[b], sems[b])
                pltpu.async_copy(dst_hbm.at[pl.ds(off, CH)], dbufs[b], sems[b])
            for b in range(KB):
                off = base + (g * KB + b) * CH
                pltpu.make_async_copy(src2_hbm.at[c].at[pl.ds(off, CH)], sbufs[b], sems[b]).wait()
                pltpu.make_async_copy(dst_hbm.at[pl.ds(off, CH)], dbufs[b], sems[b]).wait()
                pltpu.async_copy(rows_hbm.at[sbufs[b]], rbufs[b], sems[b])
            for b in range(KB):
                pltpu.make_async_copy(rows_hbm.at[sbufs[b]], rbufs[b], sems[b]).wait()
                pltpu.async_copy(rbufs[b], acc.at[dbufs[b]], sems[b], add=True)
            for b in range(KB):
                pltpu.make_async_copy(rbufs[b], acc.at[dbufs[b]], sems[b]).wait()
            return carry

        lax.fori_loop(0, n_chunks // KB, block, 0)
        plsc.subcore_barrier()

        def wcopy(i, carry):
            pltpu.sync_copy(acc.at[pl.ds(r0 + i * ZR, ZR)], zbuf_v)
            pltpu.sync_copy(zbuf_v, out_hbm.at[c].at[pl.ds(r0 + i * ZR, ZR)])
            return carry

        lax.fori_loop(0, RPT // ZR, wcopy, 0)

    return segsum_kernel


def _tc_norm_matmul(x, W1, deg_parts, R=1000):
    N, D_IN = x.shape
    D_H = W1.shape[1]
    DL = deg_parts.shape[2]

    def body(x_ref, w_ref, deg_ref, xw_ref, nrm_ref):
        d = deg_ref[...]
        out_deg = jnp.sum(d[0] + d[2], axis=1, keepdims=True) * (1.0 / DL)
        in_deg = jnp.sum(d[1] + d[3], axis=1, keepdims=True) * (1.0 / DL)
        deg = jnp.concatenate([out_deg, in_deg], axis=1)
        nrm = lax.rsqrt(jnp.maximum(deg, 1.0))
        nrm_ref[...] = nrm
        out_n = nrm[:, 0:1]
        xw_ref[...] = (
            jnp.dot(x_ref[...], w_ref[...], preferred_element_type=jnp.float32)
            * out_n
        )

    return pl.pallas_call(
        body,
        grid=(N // R,),
        in_specs=[
            pl.BlockSpec((R, D_IN), lambda i: (i, 0)),
            pl.BlockSpec((D_IN, D_H), lambda i: (0, 0)),
            pl.BlockSpec((4, R, DL), lambda i: (0, i, 0)),
        ],
        out_specs=[
            pl.BlockSpec((R, D_H), lambda i: (i, 0)),
            pl.BlockSpec((R, 2), lambda i: (i, 0)),
        ],
        out_shape=[
            jax.ShapeDtypeStruct((N, D_H), jnp.float32),
            jax.ShapeDtypeStruct((N, 2), jnp.float32),
        ],
    )(x, W1, deg_parts)


def _tc_mid(parts, norms, b1, W2p, R=1000):
    N = norms.shape[0]
    DHH = parts.shape[2]
    D_H = 2 * DHH
    D2 = W2p.shape[1]

    def body(p_ref, n_ref, b_ref, w_ref, o_ref):
        m = jnp.concatenate([p_ref[0], p_ref[1]], axis=1)
        nrm = n_ref[...]
        h = jnp.maximum(m * nrm[:, 1:2] + b_ref[...], 0.0)
        o_ref[...] = (
            jnp.dot(h, w_ref[...], preferred_element_type=jnp.float32)
            * nrm[:, 0:1]
        )

    return pl.pallas_call(
        body,
        grid=(N // R,),
        in_specs=[
            pl.BlockSpec((2, R, DHH), lambda i: (0, i, 0)),
            pl.BlockSpec((R, 2), lambda i: (i, 0)),
            pl.BlockSpec((1, D_H), lambda i: (0, 0)),
            pl.BlockSpec((D_H, D2), lambda i: (0, 0)),
        ],
        out_specs=pl.BlockSpec((R, D2), lambda i: (i, 0)),
        out_shape=jax.ShapeDtypeStruct((N, D2), jnp.float32),
    )(parts, norms, b1, W2p)


def _tc_final(parts, norms, b2p, R=1000):
    N = norms.shape[0]
    D2 = 2 * parts.shape[2]

    def body(p_ref, n_ref, b_ref, o_ref):
        m = jnp.concatenate([p_ref[0], p_ref[1]], axis=1)
        o_ref[...] = m * n_ref[:, 1:2] + b_ref[...]

    return pl.pallas_call(
        body,
        grid=(N // R,),
        in_specs=[
            pl.BlockSpec((2, R, D2 // 2), lambda i: (0, i, 0)),
            pl.BlockSpec((R, 2), lambda i: (i, 0)),
            pl.BlockSpec((1, D2), lambda i: (0, 0)),
        ],
        out_specs=pl.BlockSpec((R, D2), lambda i: (i, 0)),
        out_shape=jax.ShapeDtypeStruct((N, D2), jnp.float32),
    )(parts, norms, b2p)


def kernel(x, edge_index, W1, b1, W2, b2):
    N, D_IN = x.shape
    E = edge_index.shape[1]
    D_H = W1.shape[1]
    N_CLS = W2.shape[1]
    D2 = 64

    src = edge_index[0]
    dst = edge_index[1]
    src2 = jnp.stack([src * 2, src * 2 + 1])

    W2p = jnp.zeros((D_H, D2), jnp.float32).at[:, :N_CLS].set(W2)
    b2p = jnp.zeros((1, D2), jnp.float32).at[0, :N_CLS].set(b2)
    b1r = b1.reshape(1, D_H)

    NP = ((N + _NS * 128 - 1) // (_NS * 128)) * (_NS * 128)
    deg_const = jnp.concatenate(
        [jnp.zeros((128, _L), jnp.float32), jnp.ones((80, _L), jnp.float32)])
    deg_parts = _make_degree_kernel(N, E, NP)(src, dst, deg_const)

    xw, norms = _tc_norm_matmul(x, W1, deg_parts)

    xw2 = xw.reshape(2 * N, D_H // 2)
    cols1 = _make_segsum_kernel(E, D_H // 2, NP)(
        xw2, src2, dst, jnp.zeros((128, D_H // 2), jnp.float32))
    h2w = _tc_mid(cols1, norms, b1r, W2p)

    h2w2 = h2w.reshape(2 * N, D2 // 2)
    cols2 = _make_segsum_kernel(E, D2 // 2, NP)(
        h2w2, src2, dst, jnp.zeros((128, D2 // 2), jnp.float32))
    outp = _tc_final(cols2, norms, b2p)
    return outp[:, :N_CLS]
